# Initial kernel scaffold; baseline (speedup 1.0000x reference)
import jax
import jax.numpy as jnp
from jax.experimental import pallas as pl


def kernel(atom_from_atom, atom_from_bond, bond_from_atom, bond_from_bond, atom_segment_ids, bond_segment_ids, W_aa, b_aa, W_ab, b_ab, W_ba, b_ba, W_bb, b_bb):
    raise NotImplementedError("write your pallas kernel here")


# speedup vs baseline: 3.7386x; 3.7386x over previous
import functools

import jax
import jax.numpy as jnp
from jax import lax
from jax.experimental import pallas as pl
from jax.experimental.pallas import tpu as pltpu
from jax.experimental.pallas import tpu_sc as plsc

G = 512
FG = 85
H = 128
NA = 10000
NB = 320000

NS = 16
CB = 128
CA = 80
NCHUNKS_B = NB // CB
NCHUNKS_A = NA // CA
ITERS_B = -(-NCHUNKS_B // NS)
ITERS_A = -(-NCHUNKS_A // NS)


def _sc_body(bfa, bfb, afa, afb, bids, aids,
             out_sums, out_cntb, out_cnta,
             rowbuf, idxbuf, idxbufa, onesbuf, zbuf,
             acc_b, acc_a, cnt_b, cnt_a):
  c = lax.axis_index("c")
  s = lax.axis_index("s")

  zero16 = jnp.zeros((16,), jnp.float32)
  one16 = jnp.ones((16,), jnp.float32)

  def fill_z(i, _):
    for j in range(H // 16):
      zbuf[i, pl.ds(j * 16, 16)] = zero16
    return 0
  lax.fori_loop(0, 32, fill_z, 0)

  def fill_o(i, _):
    for j in range(H // 16):
      onesbuf[i, pl.ds(j * 16, 16)] = one16
    return 0
  lax.fori_loop(0, CB, fill_o, 0)

  pltpu.sync_copy(zbuf, acc_b.at[pl.ds(s * 32, 32)])
  pltpu.sync_copy(zbuf, acc_a.at[pl.ds(s * 32, 32)])
  pltpu.sync_copy(zbuf, cnt_b.at[pl.ds(s * 32, 32)])
  pltpu.sync_copy(zbuf, cnt_a.at[pl.ds(s * 32, 32)])
  plsc.subcore_barrier()

  def do_bonds(x_hbm, ids_hbm):
    def step(i, _):
      chunk = i * NS + s

      @pl.when(chunk < NCHUNKS_B)
      def _():
        base = chunk * CB
        pltpu.sync_copy(x_hbm.at[pl.ds(base, CB)], rowbuf)
        pltpu.sync_copy(ids_hbm.at[pl.ds(base, CB)], idxbuf)
        pltpu.sync_copy(rowbuf, acc_b.at[idxbuf], add=True)
        pltpu.sync_copy(onesbuf, cnt_b.at[idxbuf], add=True)
      return 0
    lax.fori_loop(0, ITERS_B, step, 0)

  def do_atoms(x_hbm, ids_hbm):
    def step(i, _):
      chunk = i * NS + s

      @pl.when(chunk < NCHUNKS_A)
      def _():
        base = chunk * CA
        pltpu.sync_copy(x_hbm.at[pl.ds(base, CA)], rowbuf.at[pl.ds(0, CA)])
        pltpu.sync_copy(ids_hbm.at[pl.ds(base, CA)], idxbufa)
        pltpu.sync_copy(rowbuf.at[pl.ds(0, CA)], acc_a.at[idxbufa], add=True)
        pltpu.sync_copy(onesbuf.at[pl.ds(0, CA)], cnt_a.at[idxbufa], add=True)
      return 0
    lax.fori_loop(0, ITERS_A, step, 0)

  @pl.when(c == 0)
  def _():
    do_bonds(bfa, bids)
    do_atoms(afa, aids)

  @pl.when(c == 1)
  def _():
    do_bonds(bfb, bids)
    do_atoms(afb, aids)

  plsc.subcore_barrier()

  @pl.when(c == 0)
  def _():
    @pl.when(s == 0)
    def _():
      pltpu.sync_copy(acc_b, out_sums.at[2])

    @pl.when(s == 1)
    def _():
      pltpu.sync_copy(acc_a, out_sums.at[0])

    @pl.when(s == 2)
    def _():
      pltpu.sync_copy(cnt_b, out_cntb)

    @pl.when(s == 3)
    def _():
      pltpu.sync_copy(cnt_a, out_cnta)

  @pl.when(c == 1)
  def _():
    @pl.when(s == 0)
    def _():
      pltpu.sync_copy(acc_b, out_sums.at[3])

    @pl.when(s == 1)
    def _():
      pltpu.sync_copy(acc_a, out_sums.at[1])


def _tc_body(sums_ref, cntb_ref, cnta_ref, w_ref, b_ref, out_ref):
  ca = jnp.maximum(cnta_ref[:, 0:1], 1.0)
  cb = jnp.maximum(cntb_ref[:, 0:1], 1.0)
  for k in range(4):
    cnt = ca if k < 2 else cb
    mean = sums_ref[k] / cnt
    out_ref[k] = (jnp.dot(mean, w_ref[k], preferred_element_type=jnp.float32)
                  + b_ref[pl.ds(k, 1), :])


@jax.jit
def kernel(atom_from_atom, atom_from_bond, bond_from_atom, bond_from_bond,
           atom_segment_ids, bond_segment_ids,
           W_aa, b_aa, W_ab, b_ab, W_ba, b_ba, W_bb, b_bb):
  bids = bond_segment_ids.astype(jnp.int32)
  aids = atom_segment_ids.astype(jnp.int32)

  mesh = plsc.VectorSubcoreMesh(core_axis_name="c", subcore_axis_name="s")
  sc = pl.kernel(
      _sc_body,
      out_type=(
          jax.ShapeDtypeStruct((4, G, H), jnp.float32),
          jax.ShapeDtypeStruct((G, H), jnp.float32),
          jax.ShapeDtypeStruct((G, H), jnp.float32),
      ),
      mesh=mesh,
      scratch_types=(
          pltpu.VMEM((CB, H), jnp.float32),
          pltpu.VMEM((CB,), jnp.int32),
          pltpu.VMEM((CA,), jnp.int32),
          pltpu.VMEM((CB, H), jnp.float32),
          pltpu.VMEM((32, H), jnp.float32),
          pltpu.VMEM_SHARED((G, H), jnp.float32),
          pltpu.VMEM_SHARED((G, H), jnp.float32),
          pltpu.VMEM_SHARED((G, H), jnp.float32),
          pltpu.VMEM_SHARED((G, H), jnp.float32),
      ),
  )
  sums, cntb, cnta = sc(bond_from_atom, bond_from_bond,
                        atom_from_atom, atom_from_bond, bids, aids)

  w = jnp.stack([W_aa, W_ab, W_ba, W_bb])
  b = jnp.stack([b_aa, b_ab, b_ba, b_bb])
  preds = pl.pallas_call(
      _tc_body,
      out_shape=jax.ShapeDtypeStruct((4, G, FG), jnp.float32),
  )(sums, cntb, cnta, w, b)
  return preds[0], preds[1], preds[2], preds[3]


# speedup vs baseline: 4.9425x; 1.3220x over previous
import functools

import jax
import jax.numpy as jnp
from jax import lax
from jax.experimental import pallas as pl
from jax.experimental.pallas import tpu as pltpu
from jax.experimental.pallas import tpu_sc as plsc

G = 512
FG = 85
H = 128
NA = 10000
NB = 320000

NS = 16
CB = 128
CA = 80
NCHUNKS_B = NB // CB
NCHUNKS_A = NA // CA
ITERS_B = -(-NCHUNKS_B // NS)
ITERS_A = -(-NCHUNKS_A // NS)


def _sc_body(bfa, bfb, afa, afb, bids, aids,
             out_sums, out_cntb, out_cnta,
             rowbuf, idxbuf, idxbufa, onesbuf, zbuf,
             acc_b, acc_a, cnt_b, cnt_a, in_sem, sc_sem):
  c = lax.axis_index("c")
  s = lax.axis_index("s")

  zero16 = jnp.zeros((16,), jnp.float32)
  one16 = jnp.ones((16,), jnp.float32)

  def fill_z(i, _):
    for j in range(H // 16):
      zbuf[i, pl.ds(j * 16, 16)] = zero16
    return 0
  lax.fori_loop(0, 32, fill_z, 0)

  def fill_o(i, _):
    for j in range(H // 16):
      onesbuf[i, pl.ds(j * 16, 16)] = one16
    return 0
  lax.fori_loop(0, CB, fill_o, 0)

  pltpu.sync_copy(zbuf, acc_b.at[pl.ds(s * 32, 32)])
  pltpu.sync_copy(zbuf, acc_a.at[pl.ds(s * 32, 32)])
  pltpu.sync_copy(zbuf, cnt_b.at[pl.ds(s * 32, 32)])
  pltpu.sync_copy(zbuf, cnt_a.at[pl.ds(s * 32, 32)])
  plsc.subcore_barrier()

  def do_bonds(x_hbm, ids_hbm):
    def start_in(i, p):
      @pl.when(i * NS + s < NCHUNKS_B)
      def _():
        base = (i * NS + s) * CB
        pltpu.async_copy(x_hbm.at[pl.ds(base, CB)], rowbuf.at[p], in_sem.at[p])
        pltpu.async_copy(ids_hbm.at[pl.ds(base, CB)], idxbuf.at[p],
                         in_sem.at[p])

    def wait_in(i, p):
      @pl.when(i * NS + s < NCHUNKS_B)
      def _():
        base = (i * NS + s) * CB
        pltpu.make_async_copy(x_hbm.at[pl.ds(base, CB)], rowbuf.at[p],
                              in_sem.at[p]).wait()
        pltpu.make_async_copy(ids_hbm.at[pl.ds(base, CB)], idxbuf.at[p],
                              in_sem.at[p]).wait()

    def start_scatter(i, p):
      @pl.when(i * NS + s < NCHUNKS_B)
      def _():
        pltpu.async_copy(rowbuf.at[p], acc_b.at[idxbuf.at[p]], sc_sem.at[p],
                         add=True)
        pltpu.async_copy(onesbuf, cnt_b.at[idxbuf.at[p]], sc_sem.at[p],
                         add=True)

    def wait_scatter(i, p):
      @pl.when(i * NS + s < NCHUNKS_B)
      def _():
        pltpu.make_async_copy(rowbuf.at[p], acc_b.at[idxbuf.at[p]],
                              sc_sem.at[p]).wait()
        pltpu.make_async_copy(onesbuf, cnt_b.at[idxbuf.at[p]],
                              sc_sem.at[p]).wait()

    start_in(0, 0)

    def step(j, _):
      wait_in(2 * j, 0)

      @pl.when(j > 0)
      def _():
        wait_scatter(2 * j - 1, 1)
      start_in(2 * j + 1, 1)
      start_scatter(2 * j, 0)

      wait_in(2 * j + 1, 1)
      wait_scatter(2 * j, 0)
      start_in(2 * j + 2, 0)
      start_scatter(2 * j + 1, 1)
      return 0
    half = (ITERS_B + 1) // 2
    lax.fori_loop(0, half, step, 0)
    wait_scatter(2 * half - 1, 1)

  def do_atoms(x_hbm, ids_hbm):
    def step(i, _):
      chunk = i * NS + s

      @pl.when(chunk < NCHUNKS_A)
      def _():
        base = chunk * CA
        pltpu.sync_copy(x_hbm.at[pl.ds(base, CA)], rowbuf.at[0, pl.ds(0, CA)])
        pltpu.sync_copy(ids_hbm.at[pl.ds(base, CA)], idxbufa)
        pltpu.sync_copy(rowbuf.at[0, pl.ds(0, CA)], acc_a.at[idxbufa],
                        add=True)
        pltpu.sync_copy(onesbuf.at[pl.ds(0, CA)], cnt_a.at[idxbufa], add=True)
      return 0
    lax.fori_loop(0, ITERS_A, step, 0)

  @pl.when(c == 0)
  def _():
    do_bonds(bfa, bids)
    do_atoms(afa, aids)

  @pl.when(c == 1)
  def _():
    do_bonds(bfb, bids)
    do_atoms(afb, aids)

  plsc.subcore_barrier()

  @pl.when(c == 0)
  def _():
    @pl.when(s == 0)
    def _():
      pltpu.sync_copy(acc_b, out_sums.at[2])

    @pl.when(s == 1)
    def _():
      pltpu.sync_copy(acc_a, out_sums.at[0])

    @pl.when(s == 2)
    def _():
      pltpu.sync_copy(cnt_b, out_cntb)

    @pl.when(s == 3)
    def _():
      pltpu.sync_copy(cnt_a, out_cnta)

  @pl.when(c == 1)
  def _():
    @pl.when(s == 0)
    def _():
      pltpu.sync_copy(acc_b, out_sums.at[3])

    @pl.when(s == 1)
    def _():
      pltpu.sync_copy(acc_a, out_sums.at[1])


def _tc_body(sums_ref, cntb_ref, cnta_ref, w_ref, b_ref, out_ref):
  ca = jnp.maximum(cnta_ref[:, 0:1], 1.0)
  cb = jnp.maximum(cntb_ref[:, 0:1], 1.0)
  for k in range(4):
    cnt = ca if k < 2 else cb
    mean = sums_ref[k] / cnt
    out_ref[k] = (jnp.dot(mean, w_ref[k], preferred_element_type=jnp.float32)
                  + b_ref[pl.ds(k, 1), :])


@jax.jit
def kernel(atom_from_atom, atom_from_bond, bond_from_atom, bond_from_bond,
           atom_segment_ids, bond_segment_ids,
           W_aa, b_aa, W_ab, b_ab, W_ba, b_ba, W_bb, b_bb):
  bids = bond_segment_ids.astype(jnp.int32)
  aids = atom_segment_ids.astype(jnp.int32)

  mesh = plsc.VectorSubcoreMesh(core_axis_name="c", subcore_axis_name="s")
  sc = pl.kernel(
      _sc_body,
      out_type=(
          jax.ShapeDtypeStruct((4, G, H), jnp.float32),
          jax.ShapeDtypeStruct((G, H), jnp.float32),
          jax.ShapeDtypeStruct((G, H), jnp.float32),
      ),
      mesh=mesh,
      scratch_types=(
          pltpu.VMEM((2, CB, H), jnp.float32),
          pltpu.VMEM((2, CB), jnp.int32),
          pltpu.VMEM((CA,), jnp.int32),
          pltpu.VMEM((CB, H), jnp.float32),
          pltpu.VMEM((32, H), jnp.float32),
          pltpu.VMEM_SHARED((G, H), jnp.float32),
          pltpu.VMEM_SHARED((G, H), jnp.float32),
          pltpu.VMEM_SHARED((G, H), jnp.float32),
          pltpu.VMEM_SHARED((G, H), jnp.float32),
          pltpu.SemaphoreType.DMA((2,)),
          pltpu.SemaphoreType.DMA((2,)),
      ),
  )
  sums, cntb, cnta = sc(bond_from_atom, bond_from_bond,
                        atom_from_atom, atom_from_bond, bids, aids)

  w = jnp.stack([W_aa, W_ab, W_ba, W_bb])
  b = jnp.stack([b_aa, b_ab, b_ba, b_bb])
  preds = pl.pallas_call(
      _tc_body,
      out_shape=jax.ShapeDtypeStruct((4, G, FG), jnp.float32),
  )(sums, cntb, cnta, w, b)
  return preds[0], preds[1], preds[2], preds[3]
